# Initial kernel scaffold; baseline (speedup 1.0000x reference)
import jax
import jax.numpy as jnp
from jax.experimental import pallas as pl


def kernel(x, edge_index, edge_weight, gamma, beta, W, b):
    raise NotImplementedError("write your pallas kernel here")


# speedup vs baseline: 4.9431x; 4.9431x over previous
import functools

import jax
import jax.numpy as jnp
from jax import lax
from jax.experimental import pallas as pl
from jax.experimental.pallas import tpu as pltpu
from jax.experimental.pallas import tpu_sc as plsc

N = 10000
E = 320000
D = 128
G = 32
EPS = 1e-5

NC = 2
NS = 16
L = 16

_MESH = plsc.VectorSubcoreMesh(core_axis_name="c", subcore_axis_name="s")


_C1 = 2000
_EP1 = E // (NC * NS)


@functools.partial(
    pl.kernel,
    out_type=jax.ShapeDtypeStruct((NC * 2 * N,), jnp.float32),
    mesh=_MESH,
    compiler_params=pltpu.CompilerParams(needs_layout_passes=False, use_tc_tiling_on_sc=False),
    scratch_types=[
        pltpu.VMEM((_C1,), jnp.int32),
        pltpu.VMEM((_C1,), jnp.int32),
        pltpu.VMEM((_C1,), jnp.float32),
        pltpu.VMEM((1280,), jnp.float32),
        pltpu.VMEM_SHARED((2 * N,), jnp.float32),
    ],
)
def _deg_kernel(src_hbm, dst_hbm, ew_hbm, out_hbm,
                src_v, dst_v, ew_v, zbuf, acc):
    c = lax.axis_index("c")
    s = lax.axis_index("s")

    for i in range(1280 // L):
        zbuf[pl.ds(i * L, L)] = jnp.zeros((L,), jnp.float32)

    @pl.when(s < NS - 1)
    def _():
        pltpu.sync_copy(zbuf.at[pl.ds(0, 1248)],
                        acc.at[pl.ds(s * 1248, 1248)])

    @pl.when(s == NS - 1)
    def _():
        pltpu.sync_copy(zbuf, acc.at[pl.ds((NS - 1) * 1248, 1280)])

    plsc.subcore_barrier()

    base = (c * NS + s) * _EP1

    def chunk(j, carry):
        off = base + j * _C1
        pltpu.sync_copy(src_hbm.at[pl.ds(off, _C1)], src_v)
        pltpu.sync_copy(dst_hbm.at[pl.ds(off, _C1)], dst_v)
        pltpu.sync_copy(ew_hbm.at[pl.ds(off, _C1)], ew_v)
        for i in range(_C1 // L):
            sl = pl.ds(i * L, L)
            dst_v[sl] = dst_v[sl] + N
        pltpu.sync_copy(ew_v, acc.at[src_v], add=True)
        pltpu.sync_copy(ew_v, acc.at[dst_v], add=True)
        return carry

    lax.fori_loop(0, _EP1 // _C1, chunk, 0)

    plsc.subcore_barrier()

    obase = c * 2 * N

    @pl.when(s < NS - 1)
    def _():
        off = s * 1248
        pltpu.sync_copy(acc.at[pl.ds(off, 1248)], zbuf.at[pl.ds(0, 1248)])
        pltpu.sync_copy(zbuf.at[pl.ds(0, 1248)],
                        out_hbm.at[pl.ds(obase + off, 1248)])

    @pl.when(s == NS - 1)
    def _():
        off = (NS - 1) * 1248
        pltpu.sync_copy(acc.at[pl.ds(off, 1280)], zbuf)
        pltpu.sync_copy(zbuf, out_hbm.at[pl.ds(obase + off, 1280)])


def _safe_rsqrt(deg):
    deg_safe = jnp.where(deg > 0, deg, 1.0)
    return jnp.where(deg > 0, lax.rsqrt(deg_safe), 0.0)


def _dense_body(x_ref, gamma_ref, beta_ref, w_ref, degp_ref,
                featg_ref, normdst_ref):
    x = x_ref[...]
    mean = jnp.mean(x, axis=0, keepdims=True)
    xc = x - mean
    var = jnp.mean(xc * xc, axis=0, keepdims=True)
    h = xc * lax.rsqrt(var + EPS) * gamma_ref[...] + beta_ref[...]
    h = jnp.maximum(h, 0.0)
    dsum = degp_ref[0] + degp_ref[1]
    norm_src = _safe_rsqrt(dsum[:N])
    feat = jnp.dot(h * norm_src, w_ref[...],
                   preferred_element_type=jnp.float32)
    for g in range(G // 8):
        featg_ref[g] = feat[:, g * 8:(g + 1) * 8]
    normdst_ref[...] = _safe_rsqrt(dsum[N:])


_dense_call = pl.pallas_call(
    _dense_body,
    out_shape=[
        jax.ShapeDtypeStruct((G // 8, N, 8), jnp.float32),
        jax.ShapeDtypeStruct((N, 1), jnp.float32),
    ],
)


_C2 = 800
_TPG = 8
_EP2 = E // _TPG


@functools.partial(
    pl.kernel,
    out_type=jax.ShapeDtypeStruct((NC * 2 * N, 8), jnp.float32),
    mesh=_MESH,
    compiler_params=pltpu.CompilerParams(needs_layout_passes=False, use_tc_tiling_on_sc=False),
    scratch_types=[
        pltpu.VMEM((N, 8), jnp.float32),
        pltpu.VMEM((N,), jnp.float32),
        pltpu.VMEM((_C2,), jnp.int32),
        pltpu.VMEM((_C2,), jnp.int32),
        pltpu.VMEM((_C2,), jnp.float32),
        pltpu.VMEM((_C2, 8), jnp.float32),
        pltpu.VMEM((1280, 8), jnp.float32),
        pltpu.VMEM_SHARED((2 * N, 8), jnp.float32),
    ],
)
def _agg_kernel(featg_hbm, normdst_hbm, src_hbm, dst_hbm, ew_hbm, binit_hbm,
                out_hbm, featpart_v, normdst_v, src_v, dst_v, ew_v, msg_v,
                tbuf, acc):
    c = lax.axis_index("c")
    s = lax.axis_index("s")
    grp = s // _TPG
    slot = s % _TPG

    pltpu.sync_copy(featg_hbm.at[c * 2 + grp], featpart_v)
    pltpu.sync_copy(normdst_hbm, normdst_v)

    @pl.when(s < NS - 1)
    def _():
        io = s * 1248
        pltpu.sync_copy(binit_hbm.at[c, pl.ds(io, 1248)],
                        tbuf.at[pl.ds(0, 1248)])
        pltpu.sync_copy(tbuf.at[pl.ds(0, 1248)], acc.at[pl.ds(io, 1248)])

    @pl.when(s == NS - 1)
    def _():
        io = (NS - 1) * 1248
        pltpu.sync_copy(binit_hbm.at[c, pl.ds(io, 1280)], tbuf)
        pltpu.sync_copy(tbuf, acc.at[pl.ds(io, 1280)])

    plsc.subcore_barrier()

    base = slot * _EP2
    iota16 = lax.iota(jnp.int32, L)
    grp_off = grp * N

    def chunk(j, carry):
        off = base + j * _C2
        pltpu.sync_copy(src_hbm.at[pl.ds(off, _C2)], src_v)
        pltpu.sync_copy(dst_hbm.at[pl.ds(off, _C2)], dst_v)
        pltpu.sync_copy(ew_hbm.at[pl.ds(off, _C2)], ew_v)
        for i in range(_C2 // L):
            sl = pl.ds(i * L, L)
            sv = src_v[sl]
            dv = dst_v[sl]
            w2 = ew_v[sl] * plsc.load_gather(normdst_v, [dv])
            dst_v[sl] = dv + grp_off
            rowi = iota16 + (i * L)
            for f in range(8):
                colf = jnp.full((L,), f, jnp.int32)
                vals = plsc.load_gather(featpart_v, [sv, colf]) * w2
                plsc.store_scatter(msg_v, [rowi, colf], vals)
        pltpu.sync_copy(msg_v, acc.at[dst_v], add=True)
        return carry

    lax.fori_loop(0, _EP2 // _C2, chunk, 0)

    plsc.subcore_barrier()

    obase = c * 2 * N

    @pl.when(s < NS - 1)
    def _():
        r0 = s * 624
        for half in range(2):
            a0 = half * N + r0
            pltpu.sync_copy(acc.at[pl.ds(a0, 624)], tbuf.at[pl.ds(0, 624)])
            pltpu.sync_copy(tbuf.at[pl.ds(0, 624)],
                            out_hbm.at[pl.ds(obase + a0, 624)])

    @pl.when(s == NS - 1)
    def _():
        r0 = (NS - 1) * 624
        rows = N - r0
        for half in range(2):
            a0 = half * N + r0
            pltpu.sync_copy(acc.at[pl.ds(a0, rows)], tbuf.at[pl.ds(0, rows)])
            pltpu.sync_copy(tbuf.at[pl.ds(0, rows)],
                            out_hbm.at[pl.ds(obase + a0, rows)])


def kernel(x, edge_index, edge_weight, gamma, beta, W, b):
    src = edge_index[0]
    dst = edge_index[1]
    degp = _deg_kernel(src, dst, edge_weight)
    featg, normdst = _dense_call(
        x, gamma.reshape(1, D), beta.reshape(1, D), W,
        degp.reshape(NC, 2 * N, 1))
    binit = jnp.broadcast_to(
        b.reshape(NC, 2, 1, 8), (NC, 2, N, 8)).reshape(NC, 2 * N, 8)
    outf = _agg_kernel(featg, normdst.reshape(N), src, dst, edge_weight,
                       binit)
    out4 = outf.reshape(NC, 2, N, 8)
    return jnp.concatenate(
        [out4[0, 0], out4[0, 1], out4[1, 0], out4[1, 1]], axis=-1)


# speedup vs baseline: 5.6078x; 1.1345x over previous
import functools

import jax
import jax.numpy as jnp
from jax import lax
from jax.experimental import pallas as pl
from jax.experimental.pallas import tpu as pltpu
from jax.experimental.pallas import tpu_sc as plsc

N = 10000
E = 320000
D = 128
G = 32
EPS = 1e-5

NC = 2
NS = 16
L = 16

_MESH = plsc.VectorSubcoreMesh(core_axis_name="c", subcore_axis_name="s")


_C1 = 2000
_EP1 = E // (NC * NS)


@functools.partial(
    pl.kernel,
    out_type=jax.ShapeDtypeStruct((NC * 2 * N,), jnp.float32),
    mesh=_MESH,
    compiler_params=pltpu.CompilerParams(needs_layout_passes=False, use_tc_tiling_on_sc=False),
    scratch_types=[
        pltpu.VMEM((_C1,), jnp.int32),
        pltpu.VMEM((_C1,), jnp.int32),
        pltpu.VMEM((_C1,), jnp.float32),
        pltpu.VMEM((1280,), jnp.float32),
        pltpu.VMEM_SHARED((2 * N,), jnp.float32),
    ],
)
def _deg_kernel(src_hbm, dst_hbm, ew_hbm, out_hbm,
                src_v, dst_v, ew_v, zbuf, acc):
    c = lax.axis_index("c")
    s = lax.axis_index("s")

    for i in range(1280 // L):
        zbuf[pl.ds(i * L, L)] = jnp.zeros((L,), jnp.float32)

    @pl.when(s < NS - 1)
    def _():
        pltpu.sync_copy(zbuf.at[pl.ds(0, 1248)],
                        acc.at[pl.ds(s * 1248, 1248)])

    @pl.when(s == NS - 1)
    def _():
        pltpu.sync_copy(zbuf, acc.at[pl.ds((NS - 1) * 1248, 1280)])

    plsc.subcore_barrier()

    base = (c * NS + s) * _EP1

    def chunk(j, carry):
        off = base + j * _C1
        pltpu.sync_copy(src_hbm.at[pl.ds(off, _C1)], src_v)
        pltpu.sync_copy(dst_hbm.at[pl.ds(off, _C1)], dst_v)
        pltpu.sync_copy(ew_hbm.at[pl.ds(off, _C1)], ew_v)
        for i in range(_C1 // L):
            sl = pl.ds(i * L, L)
            dst_v[sl] = dst_v[sl] + N
        pltpu.sync_copy(ew_v, acc.at[src_v], add=True)
        pltpu.sync_copy(ew_v, acc.at[dst_v], add=True)
        return carry

    lax.fori_loop(0, _EP1 // _C1, chunk, 0)

    plsc.subcore_barrier()

    obase = c * 2 * N

    @pl.when(s < NS - 1)
    def _():
        off = s * 1248
        pltpu.sync_copy(acc.at[pl.ds(off, 1248)], zbuf.at[pl.ds(0, 1248)])
        pltpu.sync_copy(zbuf.at[pl.ds(0, 1248)],
                        out_hbm.at[pl.ds(obase + off, 1248)])

    @pl.when(s == NS - 1)
    def _():
        off = (NS - 1) * 1248
        pltpu.sync_copy(acc.at[pl.ds(off, 1280)], zbuf)
        pltpu.sync_copy(zbuf, out_hbm.at[pl.ds(obase + off, 1280)])


def _safe_rsqrt(deg):
    deg_safe = jnp.where(deg > 0, deg, 1.0)
    return jnp.where(deg > 0, lax.rsqrt(deg_safe), 0.0)


def _dense_body(x_ref, gamma_ref, beta_ref, w_ref, degp_ref,
                featg_ref, normdst_ref):
    x = x_ref[...]
    mean = jnp.mean(x, axis=0, keepdims=True)
    xc = x - mean
    var = jnp.mean(xc * xc, axis=0, keepdims=True)
    h = xc * lax.rsqrt(var + EPS) * gamma_ref[...] + beta_ref[...]
    h = jnp.maximum(h, 0.0)
    dsum = degp_ref[0] + degp_ref[1]
    norm_src = _safe_rsqrt(dsum[:N])
    featg_ref[...] = lax.dot_general(
        w_ref[...], h * norm_src, (((0,), (1,)), ((), ())),
        preferred_element_type=jnp.float32)
    normdst_ref[...] = _safe_rsqrt(dsum[N:])


_dense_call = pl.pallas_call(
    _dense_body,
    out_shape=[
        jax.ShapeDtypeStruct((G, N), jnp.float32),
        jax.ShapeDtypeStruct((N, 1), jnp.float32),
    ],
)


_C2 = 2000
_TPG = 4
_FPT = 4
_EP2 = E // _TPG


@functools.partial(
    pl.kernel,
    out_type=jax.ShapeDtypeStruct((NC * NS, _FPT * N), jnp.float32),
    mesh=_MESH,
    compiler_params=pltpu.CompilerParams(needs_layout_passes=False, use_tc_tiling_on_sc=False),
    scratch_types=[
        pltpu.VMEM((_FPT * N,), jnp.float32),
        pltpu.VMEM((N,), jnp.float32),
        pltpu.VMEM((_C2,), jnp.int32),
        pltpu.VMEM((_C2,), jnp.int32),
        pltpu.VMEM((_C2,), jnp.float32),
        pltpu.VMEM((_FPT * N,), jnp.float32),
    ],
)
def _agg_kernel(featg_hbm, normdst_hbm, src_hbm, dst_hbm, ew_hbm,
                out_hbm, featpart_v, normdst_v, src_v, dst_v, ew_v, acc_v):
    c = lax.axis_index("c")
    s = lax.axis_index("s")
    fg = c * 4 + s // _TPG
    slot = s % _TPG
    t = c * NS + s

    for f in range(_FPT):
        pltpu.sync_copy(featg_hbm.at[fg * _FPT + f],
                        featpart_v.at[pl.ds(f * N, N)])
    pltpu.sync_copy(normdst_hbm, normdst_v)

    def zero(j, carry):
        for i in range(100):
            acc_v[pl.ds(j * 1600 + i * L, L)] = jnp.zeros((L,), jnp.float32)
        return carry

    lax.fori_loop(0, _FPT * N // 1600, zero, 0)

    base = slot * _EP2

    def chunk(j, carry):
        off = base + j * _C2
        pltpu.sync_copy(src_hbm.at[pl.ds(off, _C2)], src_v)
        pltpu.sync_copy(dst_hbm.at[pl.ds(off, _C2)], dst_v)
        pltpu.sync_copy(ew_hbm.at[pl.ds(off, _C2)], ew_v)
        for i in range(_C2 // L):
            sl = pl.ds(i * L, L)
            sv = src_v[sl]
            dv = dst_v[sl]
            w2 = ew_v[sl] * plsc.load_gather(normdst_v, [dv])
            for f in range(_FPT):
                vals = plsc.load_gather(featpart_v, [sv + (f * N)]) * w2
                plsc.addupdate_scatter(acc_v, [dv + (f * N)], vals)
        return carry

    lax.fori_loop(0, _EP2 // _C2, chunk, 0)

    pltpu.sync_copy(acc_v, out_hbm.at[t])


def _epi_body(p_ref, b_ref, out_ref):
    p = p_ref[...]
    q = p.reshape(G // _FPT, _TPG, _FPT, N)
    ps = q.sum(axis=1).reshape(G, N)
    out_ref[...] = ps.T + b_ref[...]


_epi_call = pl.pallas_call(
    _epi_body,
    out_shape=jax.ShapeDtypeStruct((N, G), jnp.float32),
)


def kernel(x, edge_index, edge_weight, gamma, beta, W, b):
    src = edge_index[0]
    dst = edge_index[1]
    degp = _deg_kernel(src, dst, edge_weight)
    featg, normdst = _dense_call(
        x, gamma.reshape(1, D), beta.reshape(1, D), W,
        degp.reshape(NC, 2 * N, 1))
    partials = _agg_kernel(featg, normdst.reshape(N), src, dst,
                           edge_weight)
    return _epi_call(partials, b.reshape(1, G))


# speedup vs baseline: 8.6179x; 1.5368x over previous
import functools

import jax
import jax.numpy as jnp
from jax import lax
from jax.experimental import pallas as pl
from jax.experimental.pallas import tpu as pltpu
from jax.experimental.pallas import tpu_sc as plsc

N = 10000
E = 320000
D = 128
G = 32
EPS = 1e-5

NC = 2
NS = 16
L = 16

_MESH = plsc.VectorSubcoreMesh(core_axis_name="c", subcore_axis_name="s")


_C1 = 2000
_EP1 = E // (NC * NS)


@functools.partial(
    pl.kernel,
    out_type=jax.ShapeDtypeStruct((NC * 2 * N,), jnp.float32),
    mesh=_MESH,
    compiler_params=pltpu.CompilerParams(needs_layout_passes=False, use_tc_tiling_on_sc=False),
    scratch_types=[
        pltpu.VMEM((_C1,), jnp.int32),
        pltpu.VMEM((_C1,), jnp.int32),
        pltpu.VMEM((_C1,), jnp.float32),
        pltpu.VMEM((1280,), jnp.float32),
        pltpu.VMEM_SHARED((2 * N,), jnp.float32),
    ],
)
def _deg_kernel(src_hbm, dst_hbm, ew_hbm, out_hbm,
                src_v, dst_v, ew_v, zbuf, acc):
    c = lax.axis_index("c")
    s = lax.axis_index("s")

    for i in range(1280 // L):
        zbuf[pl.ds(i * L, L)] = jnp.zeros((L,), jnp.float32)

    @pl.when(s < NS - 1)
    def _():
        pltpu.sync_copy(zbuf.at[pl.ds(0, 1248)],
                        acc.at[pl.ds(s * 1248, 1248)])

    @pl.when(s == NS - 1)
    def _():
        pltpu.sync_copy(zbuf, acc.at[pl.ds((NS - 1) * 1248, 1280)])

    plsc.subcore_barrier()

    base = (c * NS + s) * _EP1

    def chunk(j, carry):
        off = base + j * _C1
        pltpu.sync_copy(src_hbm.at[pl.ds(off, _C1)], src_v)
        pltpu.sync_copy(dst_hbm.at[pl.ds(off, _C1)], dst_v)
        pltpu.sync_copy(ew_hbm.at[pl.ds(off, _C1)], ew_v)
        for i in range(_C1 // L):
            sl = pl.ds(i * L, L)
            dst_v[sl] = dst_v[sl] + N
        pltpu.sync_copy(ew_v, acc.at[src_v], add=True)
        pltpu.sync_copy(ew_v, acc.at[dst_v], add=True)
        return carry

    lax.fori_loop(0, _EP1 // _C1, chunk, 0)

    plsc.subcore_barrier()

    obase = c * 2 * N

    @pl.when(s < NS - 1)
    def _():
        off = s * 1248
        pltpu.sync_copy(acc.at[pl.ds(off, 1248)], zbuf.at[pl.ds(0, 1248)])
        pltpu.sync_copy(zbuf.at[pl.ds(0, 1248)],
                        out_hbm.at[pl.ds(obase + off, 1248)])

    @pl.when(s == NS - 1)
    def _():
        off = (NS - 1) * 1248
        pltpu.sync_copy(acc.at[pl.ds(off, 1280)], zbuf)
        pltpu.sync_copy(zbuf, out_hbm.at[pl.ds(obase + off, 1280)])


def _safe_rsqrt(deg):
    deg_safe = jnp.where(deg > 0, deg, 1.0)
    return jnp.where(deg > 0, lax.rsqrt(deg_safe), 0.0)


def _dense_body(x_ref, gamma_ref, beta_ref, w_ref, degp_ref,
                featg_ref, normdst_ref):
    x = x_ref[...]
    mean = jnp.mean(x, axis=0, keepdims=True)
    xc = x - mean
    var = jnp.mean(xc * xc, axis=0, keepdims=True)
    h = xc * lax.rsqrt(var + EPS) * gamma_ref[...] + beta_ref[...]
    h = jnp.maximum(h, 0.0)
    dsum = degp_ref[0] + degp_ref[1]
    norm_src = _safe_rsqrt(dsum[:N])
    featg_ref[...] = lax.dot_general(
        w_ref[...], h * norm_src, (((0,), (1,)), ((), ())),
        preferred_element_type=jnp.float32)
    normdst_ref[...] = _safe_rsqrt(dsum[N:])


_dense_call = pl.pallas_call(
    _dense_body,
    out_shape=[
        jax.ShapeDtypeStruct((G, N), jnp.float32),
        jax.ShapeDtypeStruct((N, 1), jnp.float32),
    ],
)


_C2 = 2000
_TPG = 4
_FPT = 4
_EP2 = E // _TPG


@functools.partial(
    pl.kernel,
    out_type=jax.ShapeDtypeStruct((NC * NS, _FPT * N), jnp.float32),
    mesh=_MESH,
    compiler_params=pltpu.CompilerParams(needs_layout_passes=False, use_tc_tiling_on_sc=False),
    scratch_types=[
        pltpu.VMEM((_FPT * N,), jnp.float32),
        pltpu.VMEM((N,), jnp.float32),
        pltpu.VMEM((_C2,), jnp.int32),
        pltpu.VMEM((_C2,), jnp.int32),
        pltpu.VMEM((_C2,), jnp.float32),
        pltpu.VMEM((_FPT * N,), jnp.float32),
    ],
)
def _agg_kernel(featg_hbm, normdst_hbm, src_hbm, dst_hbm, ew_hbm,
                out_hbm, featpart_v, normdst_v, src_v, dst_v, ew_v, acc_v):
    c = lax.axis_index("c")
    s = lax.axis_index("s")
    fg = c * 4 + s // _TPG
    slot = s % _TPG
    t = c * NS + s

    for f in range(_FPT):
        pltpu.sync_copy(featg_hbm.at[fg * _FPT + f],
                        featpart_v.at[pl.ds(f * N, N)])
    pltpu.sync_copy(normdst_hbm, normdst_v)

    def zero(j, carry):
        for i in range(100):
            acc_v[pl.ds(j * 1600 + i * L, L)] = jnp.zeros((L,), jnp.float32)
        return carry

    lax.fori_loop(0, _FPT * N // 1600, zero, 0)

    base = slot * _EP2

    def chunk(j, carry):
        off = base + j * _C2
        pltpu.sync_copy(src_hbm.at[pl.ds(off, _C2)], src_v)
        pltpu.sync_copy(dst_hbm.at[pl.ds(off, _C2)], dst_v)
        pltpu.sync_copy(ew_hbm.at[pl.ds(off, _C2)], ew_v)

        @plsc.parallel_loop(0, _C2 // L, unroll=8)
        def _(i):
            sl = pl.ds(i * L, L)
            sv = src_v[sl]
            dv = dst_v[sl]
            w2 = ew_v[sl] * plsc.load_gather(normdst_v, [dv])
            for f in range(_FPT):
                vals = plsc.load_gather(featpart_v, [sv + (f * N)]) * w2
                plsc.addupdate_scatter(acc_v, [dv + (f * N)], vals)

        return carry

    lax.fori_loop(0, _EP2 // _C2, chunk, 0)

    pltpu.sync_copy(acc_v, out_hbm.at[t])


def _epi_body(p_ref, b_ref, out_ref):
    p = p_ref[...]
    q = p.reshape(G // _FPT, _TPG, _FPT, N)
    ps = q.sum(axis=1).reshape(G, N)
    out_ref[...] = ps.T + b_ref[...]


_epi_call = pl.pallas_call(
    _epi_body,
    out_shape=jax.ShapeDtypeStruct((N, G), jnp.float32),
)


def kernel(x, edge_index, edge_weight, gamma, beta, W, b):
    src = edge_index[0]
    dst = edge_index[1]
    degp = _deg_kernel(src, dst, edge_weight)
    featg, normdst = _dense_call(
        x, gamma.reshape(1, D), beta.reshape(1, D), W,
        degp.reshape(NC, 2 * N, 1))
    partials = _agg_kernel(featg, normdst.reshape(N), src, dst,
                           edge_weight)
    return _epi_call(partials, b.reshape(1, G))


# speedup vs baseline: 9.0472x; 1.0498x over previous
import functools

import jax
import jax.numpy as jnp
from jax import lax
from jax.experimental import pallas as pl
from jax.experimental.pallas import tpu as pltpu
from jax.experimental.pallas import tpu_sc as plsc

N = 10000
E = 320000
D = 128
G = 32
EPS = 1e-5

NC = 2
NS = 16
L = 16

_MESH = plsc.VectorSubcoreMesh(core_axis_name="c", subcore_axis_name="s")


_C1 = 2000
_EP1 = E // (NC * NS)


@functools.partial(
    pl.kernel,
    out_type=jax.ShapeDtypeStruct((NC * 2 * N,), jnp.float32),
    mesh=_MESH,
    compiler_params=pltpu.CompilerParams(needs_layout_passes=False, use_tc_tiling_on_sc=False),
    scratch_types=[
        pltpu.VMEM((_C1,), jnp.int32),
        pltpu.VMEM((_C1,), jnp.int32),
        pltpu.VMEM((_C1,), jnp.float32),
        pltpu.VMEM((N,), jnp.float32),
        pltpu.VMEM((N,), jnp.float32),
        pltpu.VMEM((1280,), jnp.float32),
        pltpu.VMEM((1280,), jnp.float32),
        pltpu.VMEM_SHARED((NS, 2 * N), jnp.float32),
    ],
)
def _deg_kernel(src_hbm, dst_hbm, ew_hbm, out_hbm,
                src_v, dst_v, ew_v, dego_v, degi_v, accbuf, tmpbuf, stage_sh):
    c = lax.axis_index("c")
    s = lax.axis_index("s")

    def zero(j, carry):
        for i in range(5):
            o = pl.ds(j * 80 + i * L, L)
            dego_v[o] = jnp.zeros((L,), jnp.float32)
            degi_v[o] = jnp.zeros((L,), jnp.float32)
        return carry

    lax.fori_loop(0, N // 80, zero, 0)

    base = (c * NS + s) * _EP1

    def chunk(j, carry):
        off = base + j * _C1
        pltpu.sync_copy(src_hbm.at[pl.ds(off, _C1)], src_v)
        pltpu.sync_copy(dst_hbm.at[pl.ds(off, _C1)], dst_v)
        pltpu.sync_copy(ew_hbm.at[pl.ds(off, _C1)], ew_v)

        @plsc.parallel_loop(0, _C1 // L, unroll=8)
        def _(i):
            sl = pl.ds(i * L, L)
            wv = ew_v[sl]
            plsc.addupdate_scatter(dego_v, [src_v[sl]], wv)
            plsc.addupdate_scatter(degi_v, [dst_v[sl]], wv)

        return carry

    lax.fori_loop(0, _EP1 // _C1, chunk, 0)

    pltpu.sync_copy(dego_v, stage_sh.at[s, pl.ds(0, N)])
    pltpu.sync_copy(degi_v, stage_sh.at[s, pl.ds(N, N)])

    plsc.subcore_barrier()

    def reduce_drain(off, rows):
        for i in range(rows // L):
            accbuf[pl.ds(i * L, L)] = jnp.zeros((L,), jnp.float32)
        for k in range(NS):
            pltpu.sync_copy(stage_sh.at[k, pl.ds(off, rows)],
                            tmpbuf.at[pl.ds(0, rows)])
            for i in range(rows // L):
                sl = pl.ds(i * L, L)
                accbuf[sl] = accbuf[sl] + tmpbuf[sl]
        pltpu.sync_copy(accbuf.at[pl.ds(0, rows)],
                        out_hbm.at[pl.ds(c * 2 * N + off, rows)])

    @pl.when(s < NS - 1)
    def _():
        reduce_drain(s * 1248, 1248)

    @pl.when(s == NS - 1)
    def _():
        reduce_drain((NS - 1) * 1248, 1280)


def _safe_rsqrt(deg):
    deg_safe = jnp.where(deg > 0, deg, 1.0)
    return jnp.where(deg > 0, lax.rsqrt(deg_safe), 0.0)


def _dense_body(x_ref, gamma_ref, beta_ref, w_ref, degp_ref,
                featg_ref, normdst_ref):
    x = x_ref[...]
    mean = jnp.mean(x, axis=0, keepdims=True)
    xc = x - mean
    var = jnp.mean(xc * xc, axis=0, keepdims=True)
    h = xc * lax.rsqrt(var + EPS) * gamma_ref[...] + beta_ref[...]
    h = jnp.maximum(h, 0.0)
    dsum = jnp.sum(degp_ref[...], axis=0)
    norm_src = _safe_rsqrt(dsum[:N])
    featg_ref[...] = lax.dot_general(
        w_ref[...], h * norm_src, (((0,), (1,)), ((), ())),
        preferred_element_type=jnp.float32)
    normdst_ref[...] = _safe_rsqrt(dsum[N:])


_dense_call = pl.pallas_call(
    _dense_body,
    out_shape=[
        jax.ShapeDtypeStruct((G, N), jnp.float32),
        jax.ShapeDtypeStruct((N, 1), jnp.float32),
    ],
)


_C2 = 2000
_TPG = 4
_FPT = 4
_EP2 = E // _TPG


@functools.partial(
    pl.kernel,
    out_type=jax.ShapeDtypeStruct((NC * NS, _FPT * N), jnp.float32),
    mesh=_MESH,
    compiler_params=pltpu.CompilerParams(needs_layout_passes=False, use_tc_tiling_on_sc=False),
    scratch_types=[
        [pltpu.VMEM((N,), jnp.float32)] * _FPT,
        [pltpu.VMEM((N,), jnp.float32)] * _FPT,
        pltpu.VMEM((_C2,), jnp.int32),
        pltpu.VMEM((_C2,), jnp.int32),
        pltpu.VMEM((_C2,), jnp.float32),
    ],
)
def _agg_kernel(featg_hbm, src_hbm, dst_hbm, ew_hbm,
                out_hbm, feat_fs, acc_fs, src_v, dst_v, ew_v):
    c = lax.axis_index("c")
    s = lax.axis_index("s")
    fg = c * 4 + s // _TPG
    slot = s % _TPG
    t = c * NS + s

    for f in range(_FPT):
        pltpu.sync_copy(featg_hbm.at[fg * _FPT + f], feat_fs[f])

    def zero(j, carry):
        for f in range(_FPT):
            acc_fs[f][pl.ds(j * L, L)] = jnp.zeros((L,), jnp.float32)
        return carry

    lax.fori_loop(0, N // L, zero, 0)

    base = slot * _EP2

    def chunk(j, carry):
        off = base + j * _C2
        pltpu.sync_copy(src_hbm.at[pl.ds(off, _C2)], src_v)
        pltpu.sync_copy(dst_hbm.at[pl.ds(off, _C2)], dst_v)
        pltpu.sync_copy(ew_hbm.at[pl.ds(off, _C2)], ew_v)

        @plsc.parallel_loop(0, _C2 // L, unroll=8)
        def _(i):
            sl = pl.ds(i * L, L)
            sv = src_v[sl]
            dv = dst_v[sl]
            wv = ew_v[sl]
            for f in range(_FPT):
                vals = plsc.load_gather(feat_fs[f], [sv]) * wv
                plsc.addupdate_scatter(acc_fs[f], [dv], vals)

        return carry

    lax.fori_loop(0, _EP2 // _C2, chunk, 0)

    for f in range(_FPT):
        pltpu.sync_copy(acc_fs[f], out_hbm.at[t, pl.ds(f * N, N)])


def _epi_body(p_ref, normdst_ref, b_ref, out_ref):
    p = p_ref[...]
    q = p.reshape(G // _FPT, _TPG, _FPT, N)
    ps = q.sum(axis=1).reshape(G, N)
    out_ref[...] = ps.T * normdst_ref[...] + b_ref[...]


_epi_call = pl.pallas_call(
    _epi_body,
    out_shape=jax.ShapeDtypeStruct((N, G), jnp.float32),
)


def kernel(x, edge_index, edge_weight, gamma, beta, W, b):
    src = edge_index[0]
    dst = edge_index[1]
    degp = _deg_kernel(src, dst, edge_weight)
    featg, normdst = _dense_call(
        x, gamma.reshape(1, D), beta.reshape(1, D), W,
        degp.reshape(NC, 2 * N, 1))
    partials = _agg_kernel(featg, src, dst, edge_weight)
    return _epi_call(partials, normdst, b.reshape(1, G))


# speedup vs baseline: 10.3034x; 1.1388x over previous
import functools

import jax
import jax.numpy as jnp
from jax import lax
from jax.experimental import pallas as pl
from jax.experimental.pallas import tpu as pltpu
from jax.experimental.pallas import tpu_sc as plsc

N = 10000
E = 320000
D = 128
G = 32
EPS = 1e-5

NC = 2
NS = 16
L = 16

_MESH = plsc.VectorSubcoreMesh(core_axis_name="c", subcore_axis_name="s")


_C1 = 2000
_EP1 = E // (NC * NS)


@functools.partial(
    pl.kernel,
    out_type=jax.ShapeDtypeStruct((NC * 2 * N,), jnp.float32),
    mesh=_MESH,
    compiler_params=pltpu.CompilerParams(needs_layout_passes=False, use_tc_tiling_on_sc=False),
    scratch_types=[
        pltpu.VMEM((_C1,), jnp.int32),
        pltpu.VMEM((_C1,), jnp.int32),
        pltpu.VMEM((_C1,), jnp.float32),
        pltpu.VMEM((N,), jnp.float32),
        pltpu.VMEM((N,), jnp.float32),
        pltpu.VMEM((1280,), jnp.float32),
        pltpu.VMEM((1280,), jnp.float32),
        pltpu.VMEM_SHARED((NS, 2 * N), jnp.float32),
    ],
)
def _deg_kernel(src_hbm, dst_hbm, ew_hbm, out_hbm,
                src_v, dst_v, ew_v, dego_v, degi_v, accbuf, tmpbuf, stage_sh):
    c = lax.axis_index("c")
    s = lax.axis_index("s")

    def zero(j, carry):
        for i in range(5):
            o = pl.ds(j * 80 + i * L, L)
            dego_v[o] = jnp.zeros((L,), jnp.float32)
            degi_v[o] = jnp.zeros((L,), jnp.float32)
        return carry

    lax.fori_loop(0, N // 80, zero, 0)

    base = (c * NS + s) * _EP1

    def chunk(j, carry):
        off = base + j * _C1
        pltpu.sync_copy(src_hbm.at[pl.ds(off, _C1)], src_v)
        pltpu.sync_copy(dst_hbm.at[pl.ds(off, _C1)], dst_v)
        pltpu.sync_copy(ew_hbm.at[pl.ds(off, _C1)], ew_v)

        @plsc.parallel_loop(0, _C1 // L, unroll=8)
        def _(i):
            sl = pl.ds(i * L, L)
            wv = ew_v[sl]
            plsc.addupdate_scatter(dego_v, [src_v[sl]], wv)
            plsc.addupdate_scatter(degi_v, [dst_v[sl]], wv)

        return carry

    lax.fori_loop(0, _EP1 // _C1, chunk, 0)

    pltpu.sync_copy(dego_v, stage_sh.at[s, pl.ds(0, N)])
    pltpu.sync_copy(degi_v, stage_sh.at[s, pl.ds(N, N)])

    plsc.subcore_barrier()

    def reduce_drain(off, rows):
        for i in range(rows // L):
            accbuf[pl.ds(i * L, L)] = jnp.zeros((L,), jnp.float32)
        for k in range(NS):
            pltpu.sync_copy(stage_sh.at[k, pl.ds(off, rows)],
                            tmpbuf.at[pl.ds(0, rows)])
            for i in range(rows // L):
                sl = pl.ds(i * L, L)
                accbuf[sl] = accbuf[sl] + tmpbuf[sl]
        pltpu.sync_copy(accbuf.at[pl.ds(0, rows)],
                        out_hbm.at[pl.ds(c * 2 * N + off, rows)])

    @pl.when(s < NS - 1)
    def _():
        reduce_drain(s * 1248, 1248)

    @pl.when(s == NS - 1)
    def _():
        reduce_drain((NS - 1) * 1248, 1280)


def _safe_rsqrt(deg):
    deg_safe = jnp.where(deg > 0, deg, 1.0)
    return jnp.where(deg > 0, lax.rsqrt(deg_safe), 0.0)


def _dense_body(x_ref, gamma_ref, beta_ref, w_ref, degp_ref,
                featg_ref, normdst_ref):
    x = x_ref[...]
    mean = jnp.mean(x, axis=0, keepdims=True)
    xc = x - mean
    var = jnp.mean(xc * xc, axis=0, keepdims=True)
    h = xc * lax.rsqrt(var + EPS) * gamma_ref[...] + beta_ref[...]
    h = jnp.maximum(h, 0.0)
    dsum = jnp.sum(degp_ref[...], axis=0)
    norm_src = _safe_rsqrt(dsum[:N])
    featg_ref[...] = lax.dot_general(
        w_ref[...], h * norm_src, (((0,), (1,)), ((), ())),
        preferred_element_type=jnp.float32)
    normdst_ref[...] = _safe_rsqrt(dsum[N:])


_dense_call = pl.pallas_call(
    _dense_body,
    out_shape=[
        jax.ShapeDtypeStruct((G, N), jnp.float32),
        jax.ShapeDtypeStruct((N, 1), jnp.float32),
    ],
)


_C2 = 4000
_TPG = 4
_FPT = 4
_EP2 = E // _TPG


@functools.partial(
    pl.kernel,
    out_type=jax.ShapeDtypeStruct((NC * NS, _FPT * N), jnp.float32),
    mesh=_MESH,
    compiler_params=pltpu.CompilerParams(needs_layout_passes=False, use_tc_tiling_on_sc=False),
    scratch_types=[
        [pltpu.VMEM((N,), jnp.float32)] * _FPT,
        [pltpu.VMEM((N,), jnp.float32)] * _FPT,
        pltpu.VMEM((_C2,), jnp.int32),
        pltpu.VMEM((_C2,), jnp.int32),
        pltpu.VMEM((_C2,), jnp.float32),
    ],
)
def _agg_kernel(featg_hbm, src_hbm, dst_hbm, ew_hbm,
                out_hbm, feat_fs, acc_fs, src_v, dst_v, ew_v):
    c = lax.axis_index("c")
    s = lax.axis_index("s")
    fg = c * 4 + s // _TPG
    slot = s % _TPG
    t = c * NS + s

    for f in range(_FPT):
        pltpu.sync_copy(featg_hbm.at[fg * _FPT + f], feat_fs[f])

    def zero(j, carry):
        for f in range(_FPT):
            acc_fs[f][pl.ds(j * L, L)] = jnp.zeros((L,), jnp.float32)
        return carry

    lax.fori_loop(0, N // L, zero, 0)

    base = slot * _EP2

    def chunk(j, carry):
        off = base + j * _C2
        pltpu.sync_copy(src_hbm.at[pl.ds(off, _C2)], src_v)
        pltpu.sync_copy(dst_hbm.at[pl.ds(off, _C2)], dst_v)
        pltpu.sync_copy(ew_hbm.at[pl.ds(off, _C2)], ew_v)

        @plsc.parallel_loop(0, _C2 // L, unroll=16)
        def _(i):
            sl = pl.ds(i * L, L)
            sv = src_v[sl]
            dv = dst_v[sl]
            wv = ew_v[sl]
            for f in range(_FPT):
                vals = plsc.load_gather(feat_fs[f], [sv]) * wv
                plsc.addupdate_scatter(acc_fs[f], [dv], vals)

        return carry

    lax.fori_loop(0, _EP2 // _C2, chunk, 0)

    for f in range(_FPT):
        pltpu.sync_copy(acc_fs[f], out_hbm.at[t, pl.ds(f * N, N)])


def _epi_body(p_ref, normdst_ref, b_ref, out_ref):
    p = p_ref[...]
    q = p.reshape(G // _FPT, _TPG, _FPT, N)
    ps = q.sum(axis=1).reshape(G, N)
    out_ref[...] = ps.T * normdst_ref[...] + b_ref[...]


_epi_call = pl.pallas_call(
    _epi_body,
    out_shape=jax.ShapeDtypeStruct((N, G), jnp.float32),
)


def kernel(x, edge_index, edge_weight, gamma, beta, W, b):
    src = edge_index[0]
    dst = edge_index[1]
    degp = _deg_kernel(src, dst, edge_weight)
    featg, normdst = _dense_call(
        x, gamma.reshape(1, D), beta.reshape(1, D), W,
        degp.reshape(NC, 2 * N, 1))
    partials = _agg_kernel(featg, src, dst, edge_weight)
    return _epi_call(partials, normdst, b.reshape(1, G))
